# Initial kernel scaffold; baseline (speedup 1.0000x reference)
import jax
import jax.numpy as jnp
from jax.experimental import pallas as pl


def kernel(edge_index, edge_values, user_emb, item_emb):
    raise NotImplementedError("write your pallas kernel here")


# speedup vs baseline: 2.3638x; 2.3638x over previous
import functools

import jax
import jax.numpy as jnp
from jax import lax
from jax.experimental import pallas as pl
from jax.experimental.pallas import tpu as pltpu
from jax.experimental.pallas import tpu_sc as plsc

N_USERS = 5000
N_ITEMS = 5000
N_NODES = N_USERS + N_ITEMS
N_EDGES = 160000
D = 256
DH = 128
N_LAYERS = 3

NC = 2
NS = 16
B = 128
G = 8
NG = 10
NB = G * NG
EPT = NB * B
E_PAD = EPT * NS
ACC_ROWS = 10112
ZPT = ACC_ROWS // NS
ZCH = 104
ROWS_PER_TILE = 624
REM_BASE = NS * ROWS_PER_TILE
REM = 8


def _sc_kernel(x0, src2, dst2, val2, out, e1, e2, e3,
               gidx_v, dst_v, val_v, rows_v, zbuf_v, acc, sem):
    c = lax.axis_index("c")
    s = lax.axis_index("s")
    base = s * NB

    off = c * N_NODES

    zero16 = jnp.zeros((16,), jnp.float32)

    def zfill(i, _):
        r = i // 8
        col = (i % 8) * 16
        zbuf_v[r, pl.ds(col, 16)] = zero16
        return 0

    lax.fori_loop(0, ZCH * 8, zfill, 0)

    def layer(ein, eout):
        for z in range(ZPT // ZCH):
            pltpu.sync_copy(zbuf_v, acc.at[pl.ds(s * ZPT + z * ZCH, ZCH)])
        pltpu.sync_copy(zbuf_v.at[pl.ds(0, ZPT % ZCH)],
                        acc.at[pl.ds(s * ZPT + (ZPT // ZCH) * ZCH, ZPT % ZCH)])
        plsc.subcore_barrier()

        def group(g, _):
            gb = base + g * G
            pltpu.sync_copy(src2.at[pl.ds(gb, G)], gidx_v)
            pltpu.sync_copy(dst2.at[pl.ds(gb, G)], dst_v)
            pltpu.sync_copy(val2.at[pl.ds(gb, G)], val_v)

            def add_off(i, _2):
                r = i // 8
                col = (i % 8) * 16
                gidx_v[r, pl.ds(col, 16)] = gidx_v[r, pl.ds(col, 16)] + off
                return 0

            lax.fori_loop(0, G * 8, add_off, 0)

            def batch(b, _2):
                pltpu.async_copy(ein.at[gidx_v.at[b]], rows_v, sem).wait()

                def scale(m, _3):
                    vals = val_v[b, pl.ds(m * 16, 16)]
                    ebase = m * 16
                    for e in range(16):
                        v = vals[e]
                        row = ebase + e
                        for j in range(8):
                            rows_v[row, pl.ds(j * 16, 16)] = (
                                rows_v[row, pl.ds(j * 16, 16)] * v)
                    return 0

                lax.fori_loop(0, B // 16, scale, 0)
                pltpu.sync_copy(rows_v, acc.at[dst_v.at[b]], add=True)
                return 0

            lax.fori_loop(0, G, batch, 0)
            return 0

        lax.fori_loop(0, NG, group, 0)
        plsc.subcore_barrier()
        pltpu.sync_copy(acc.at[pl.ds(s * ROWS_PER_TILE, ROWS_PER_TILE)],
                        eout.at[pl.ds(c * N_NODES + s * ROWS_PER_TILE, ROWS_PER_TILE)])

        @pl.when(s < 2)
        def _copy_rem():
            pltpu.sync_copy(acc.at[pl.ds(REM_BASE + s * REM, REM)],
                            eout.at[pl.ds(c * N_NODES + REM_BASE + s * REM, REM)])

        plsc.subcore_barrier()

    layer(x0, e1)
    layer(e1, e2)
    layer(e2, e3)

    def mean_chunk(rs, ch):
        pltpu.sync_copy(x0.at[pl.ds(rs, ch)], rows_v.at[pl.ds(0, ch)])
        for eh in (e1, e2, e3):
            pltpu.sync_copy(eh.at[pl.ds(rs, ch)], zbuf_v.at[pl.ds(0, ch)])

            def accum(i, _):
                r = i // 8
                col = (i % 8) * 16
                rows_v[r, pl.ds(col, 16)] = (rows_v[r, pl.ds(col, 16)]
                                             + zbuf_v[r, pl.ds(col, 16)])
                return 0

            lax.fori_loop(0, ch * 8, accum, 0)

        def scalemean(i, _):
            r = i // 8
            col = (i % 8) * 16
            rows_v[r, pl.ds(col, 16)] = rows_v[r, pl.ds(col, 16)] * 0.25
            return 0

        lax.fori_loop(0, ch * 8, scalemean, 0)
        pltpu.sync_copy(rows_v.at[pl.ds(0, ch)], out.at[pl.ds(rs, ch)])

    r0 = c * N_NODES + s * ROWS_PER_TILE
    for q in range(ROWS_PER_TILE // ZCH):
        mean_chunk(r0 + q * ZCH, ZCH)

    @pl.when(s < 2)
    def _mean_rem():
        mean_chunk(c * N_NODES + REM_BASE + s * REM, REM)


_mesh = plsc.VectorSubcoreMesh(core_axis_name="c", subcore_axis_name="s")

_f32 = jnp.float32
_i32 = jnp.int32

_kernel_call = functools.partial(
    pl.kernel,
    mesh=_mesh,
    out_type=[jax.ShapeDtypeStruct((NC * N_NODES, DH), _f32)] * 4,
    scratch_types=[
        pltpu.VMEM((G, B), _i32),
        pltpu.VMEM((G, B), _i32),
        pltpu.VMEM((G, B), _f32),
        pltpu.VMEM((B, DH), _f32),
        pltpu.VMEM((ZCH, DH), _f32),
        pltpu.VMEM_SHARED((ACC_ROWS, DH), _f32),
        pltpu.SemaphoreType.DMA,
    ],
)(_sc_kernel)


def kernel(edge_index, edge_values, user_emb, item_emb):
    ego = jnp.concatenate([user_emb, item_emb], axis=0)
    x0 = ego.reshape(N_NODES, NC, DH).transpose(1, 0, 2).reshape(NC * N_NODES, DH)

    pad = E_PAD - N_EDGES
    src = jnp.pad(edge_index[0], (0, pad))
    dst = jnp.pad(edge_index[1], (0, pad), constant_values=N_NODES)
    val = jnp.pad(edge_values, (0, pad))

    src2 = src.reshape(E_PAD // B, B)
    dst2 = dst.reshape(E_PAD // B, B)
    val2 = val.reshape(E_PAD // B, B)

    mean_cs, _e1, _e2, _e3 = _kernel_call(x0, src2, dst2, val2)
    mean = mean_cs.reshape(NC, N_NODES, DH).transpose(1, 0, 2).reshape(N_NODES, D)
    return (mean[:N_USERS], mean[N_USERS:])


# speedup vs baseline: 2.9428x; 1.2449x over previous
import functools

import jax
import jax.numpy as jnp
from jax import lax
from jax.experimental import pallas as pl
from jax.experimental.pallas import tpu as pltpu
from jax.experimental.pallas import tpu_sc as plsc

N_USERS = 5000
N_ITEMS = 5000
N_NODES = N_USERS + N_ITEMS
N_EDGES = 160000
D = 256
DH = 128

NC = 2
NS = 16
B = 128
G = 40
NG = 2
NB = G * NG
EPT = NB * B
E_PAD = EPT * NS
ACC_ROWS = 10112
ZPT = ACC_ROWS // NS
ROWS_PER_TILE = 624
REM_BASE = NS * ROWS_PER_TILE
REM = 8


def _sc_kernel(x0, src2, dst2, val2, out, e1, e2, e3,
               gidx_v, dst_v, val_v, rows_a, rows_b, acc, sem_a, sem_b):
    c = lax.axis_index("c")
    s = lax.axis_index("s")
    base = s * NB
    off = c * N_NODES
    bufs = (rows_a, rows_b)
    sems = (sem_a, sem_b)

    zero16 = jnp.zeros((16,), jnp.float32)

    def zero_rows_a():
        def zf(i, _):
            r = i // 8
            col = (i % 8) * 16
            rows_a[r, pl.ds(col, 16)] = zero16
            return 0

        lax.fori_loop(0, B * 8, zf, 0)

    def scale(buf, b):
        def sc(m, _):
            vals = val_v[b, pl.ds(m * 16, 16)]
            ebase = m * 16
            for e in range(16):
                v = vals[e]
                row = ebase + e
                for j in range(8):
                    buf[row, pl.ds(j * 16, 16)] = buf[row, pl.ds(j * 16, 16)] * v
            return 0

        lax.fori_loop(0, B // 16, sc, 0)

    def layer(ein, eout):
        zero_rows_a()
        for z in range(ZPT // B):
            pltpu.sync_copy(rows_a, acc.at[pl.ds(s * ZPT + z * B, B)])
        pltpu.sync_copy(rows_a.at[pl.ds(0, ZPT % B)],
                        acc.at[pl.ds(s * ZPT + (ZPT // B) * B, ZPT % B)])
        plsc.subcore_barrier()

        for g in range(NG):
            gb = base + g * G
            pltpu.sync_copy(src2.at[pl.ds(gb, G)], gidx_v)
            pltpu.sync_copy(dst2.at[pl.ds(gb, G)], dst_v)
            pltpu.sync_copy(val2.at[pl.ds(gb, G)], val_v)

            def add_off(i, _):
                r = i // 8
                col = (i % 8) * 16
                gidx_v[r, pl.ds(col, 16)] = gidx_v[r, pl.ds(col, 16)] + off
                return 0

            lax.fori_loop(0, G * 8, add_off, 0)

            pltpu.async_copy(ein.at[gidx_v.at[0]], rows_a, sem_a)

            def pair(k, _):
                for j in range(2):
                    b = k * 2 + j
                    buf, sem = bufs[j], sems[j]
                    obuf, osem = bufs[1 - j], sems[1 - j]
                    pltpu.make_async_copy(ein.at[pl.ds(0, B)], buf, sem).wait()

                    @pl.when(b + 1 < G)
                    def _prefetch():
                        pltpu.async_copy(ein.at[gidx_v.at[b + 1]], obuf, osem)

                    scale(buf, b)
                    pltpu.sync_copy(buf, acc.at[dst_v.at[b]], add=True)
                return 0

            lax.fori_loop(0, G // 2, pair, 0)

        plsc.subcore_barrier()
        pltpu.sync_copy(acc.at[pl.ds(s * ROWS_PER_TILE, ROWS_PER_TILE)],
                        eout.at[pl.ds(c * N_NODES + s * ROWS_PER_TILE, ROWS_PER_TILE)])

        @pl.when(s < 2)
        def _copy_rem():
            pltpu.sync_copy(acc.at[pl.ds(REM_BASE + s * REM, REM)],
                            eout.at[pl.ds(c * N_NODES + REM_BASE + s * REM, REM)])

        plsc.subcore_barrier()

    layer(x0, e1)
    layer(e1, e2)
    layer(e2, e3)

    def mean_chunk(rs, ch):
        pltpu.sync_copy(x0.at[pl.ds(rs, ch)], rows_a.at[pl.ds(0, ch)])
        for eh in (e1, e2, e3):
            pltpu.sync_copy(eh.at[pl.ds(rs, ch)], rows_b.at[pl.ds(0, ch)])

            def accum(i, _):
                r = i // 8
                col = (i % 8) * 16
                rows_a[r, pl.ds(col, 16)] = (rows_a[r, pl.ds(col, 16)]
                                             + rows_b[r, pl.ds(col, 16)])
                return 0

            lax.fori_loop(0, ch * 8, accum, 0)

        def scalemean(i, _):
            r = i // 8
            col = (i % 8) * 16
            rows_a[r, pl.ds(col, 16)] = rows_a[r, pl.ds(col, 16)] * 0.25
            return 0

        lax.fori_loop(0, ch * 8, scalemean, 0)
        pltpu.sync_copy(rows_a.at[pl.ds(0, ch)], out.at[pl.ds(rs, ch)])

    r0 = c * N_NODES + s * ROWS_PER_TILE
    for q in range(ROWS_PER_TILE // B):
        mean_chunk(r0 + q * B, B)
    mean_chunk(r0 + (ROWS_PER_TILE // B) * B, ROWS_PER_TILE % B)

    @pl.when(s < 2)
    def _mean_rem():
        mean_chunk(c * N_NODES + REM_BASE + s * REM, REM)


_mesh = plsc.VectorSubcoreMesh(core_axis_name="c", subcore_axis_name="s")

_f32 = jnp.float32
_i32 = jnp.int32

_kernel_call = functools.partial(
    pl.kernel,
    mesh=_mesh,
    out_type=[jax.ShapeDtypeStruct((NC * N_NODES, DH), _f32)] * 4,
    scratch_types=[
        pltpu.VMEM((G, B), _i32),
        pltpu.VMEM((G, B), _i32),
        pltpu.VMEM((G, B), _f32),
        pltpu.VMEM((B, DH), _f32),
        pltpu.VMEM((B, DH), _f32),
        pltpu.VMEM_SHARED((ACC_ROWS, DH), _f32),
        pltpu.SemaphoreType.DMA,
        pltpu.SemaphoreType.DMA,
    ],
)(_sc_kernel)


def kernel(edge_index, edge_values, user_emb, item_emb):
    ego = jnp.concatenate([user_emb, item_emb], axis=0)
    x0 = ego.reshape(N_NODES, NC, DH).transpose(1, 0, 2).reshape(NC * N_NODES, DH)

    pad = E_PAD - N_EDGES
    src = jnp.pad(edge_index[0], (0, pad))
    dst = jnp.pad(edge_index[1], (0, pad), constant_values=N_NODES)
    val = jnp.pad(edge_values, (0, pad))

    src2 = src.reshape(E_PAD // B, B)
    dst2 = dst.reshape(E_PAD // B, B)
    val2 = val.reshape(E_PAD // B, B)

    mean_cs, _e1, _e2, _e3 = _kernel_call(x0, src2, dst2, val2)
    mean = mean_cs.reshape(NC, N_NODES, DH).transpose(1, 0, 2).reshape(N_NODES, D)
    return (mean[:N_USERS], mean[N_USERS:])


# speedup vs baseline: 2.9478x; 1.0017x over previous
import functools

import jax
import jax.numpy as jnp
from jax import lax
from jax.experimental import pallas as pl
from jax.experimental.pallas import tpu as pltpu
from jax.experimental.pallas import tpu_sc as plsc

N_USERS = 5000
N_ITEMS = 5000
N_NODES = N_USERS + N_ITEMS
N_EDGES = 160000
D = 256
DH = 128

NC = 2
NS = 16
B = 128
G = 40
NG = 2
NB = G * NG
EPT = NB * B
E_PAD = EPT * NS
ACC_ROWS = 10112
ZPT = ACC_ROWS // NS
ROWS_PER_TILE = 624
REM_BASE = NS * ROWS_PER_TILE
REM = 8


def _sc_kernel(x0, src2, dst2, val2, out, e1, e2, e3,
               gidx_v, dst_v, val_v, rows_a, rows_b, acc,
               sem_a, sem_a2, sem_b, sem_b2):
    c = lax.axis_index("c")
    s = lax.axis_index("s")
    base = s * NB
    off = c * N_NODES
    bufs = (rows_a, rows_b)
    sems2 = ((sem_a, sem_a2), (sem_b, sem_b2))

    zero16 = jnp.zeros((16,), jnp.float32)

    def zero_rows_a():
        def zf(i, _):
            r = i // 8
            col = (i % 8) * 16
            rows_a[r, pl.ds(col, 16)] = zero16
            return 0

        lax.fori_loop(0, B * 8, zf, 0)

    def scale(buf, b):
        def sc(m, _):
            vals = val_v[b, pl.ds(m * 16, 16)]
            ebase = m * 16
            for e in range(16):
                v = vals[e]
                row = ebase + e
                for j in range(8):
                    buf[row, pl.ds(j * 16, 16)] = buf[row, pl.ds(j * 16, 16)] * v
            return 0

        lax.fori_loop(0, B // 16, sc, 0)

    def layer(ein, eout):
        zero_rows_a()
        for z in range(ZPT // B):
            pltpu.sync_copy(rows_a, acc.at[pl.ds(s * ZPT + z * B, B)])
        pltpu.sync_copy(rows_a.at[pl.ds(0, ZPT % B)],
                        acc.at[pl.ds(s * ZPT + (ZPT // B) * B, ZPT % B)])
        plsc.subcore_barrier()

        for g in range(NG):
            gb = base + g * G
            pltpu.sync_copy(src2.at[pl.ds(gb, G)], gidx_v)
            pltpu.sync_copy(dst2.at[pl.ds(gb, G)], dst_v)
            pltpu.sync_copy(val2.at[pl.ds(gb, G)], val_v)

            def add_off(i, _):
                r = i // 8
                col = (i % 8) * 16
                gidx_v[r, pl.ds(col, 16)] = gidx_v[r, pl.ds(col, 16)] + off
                return 0

            lax.fori_loop(0, G * 8, add_off, 0)

            H = B // 2
            pltpu.async_copy(ein.at[gidx_v.at[0, pl.ds(0, H)]],
                             rows_a.at[pl.ds(0, H)], sem_a)
            pltpu.async_copy(ein.at[gidx_v.at[0, pl.ds(H, H)]],
                             rows_a.at[pl.ds(H, H)], sem_a2)

            def pair(k, _):
                for j in range(2):
                    b = k * 2 + j
                    buf = bufs[j]
                    sem, sem2 = sems2[j]
                    obuf = bufs[1 - j]
                    osem, osem2 = sems2[1 - j]
                    pltpu.make_async_copy(ein.at[pl.ds(0, H)],
                                          buf.at[pl.ds(0, H)], sem).wait()
                    pltpu.make_async_copy(ein.at[pl.ds(0, H)],
                                          buf.at[pl.ds(H, H)], sem2).wait()

                    @pl.when(b + 1 < G)
                    def _prefetch():
                        pltpu.async_copy(ein.at[gidx_v.at[b + 1, pl.ds(0, H)]],
                                         obuf.at[pl.ds(0, H)], osem)
                        pltpu.async_copy(ein.at[gidx_v.at[b + 1, pl.ds(H, H)]],
                                         obuf.at[pl.ds(H, H)], osem2)

                    scale(buf, b)
                    pltpu.sync_copy(buf, acc.at[dst_v.at[b]], add=True)
                return 0

            lax.fori_loop(0, G // 2, pair, 0)

        plsc.subcore_barrier()
        pltpu.sync_copy(acc.at[pl.ds(s * ROWS_PER_TILE, ROWS_PER_TILE)],
                        eout.at[pl.ds(c * N_NODES + s * ROWS_PER_TILE, ROWS_PER_TILE)])

        @pl.when(s < 2)
        def _copy_rem():
            pltpu.sync_copy(acc.at[pl.ds(REM_BASE + s * REM, REM)],
                            eout.at[pl.ds(c * N_NODES + REM_BASE + s * REM, REM)])

        plsc.subcore_barrier()

    layer(x0, e1)
    layer(e1, e2)
    layer(e2, e3)

    def mean_chunk(rs, ch):
        pltpu.sync_copy(x0.at[pl.ds(rs, ch)], rows_a.at[pl.ds(0, ch)])
        for eh in (e1, e2, e3):
            pltpu.sync_copy(eh.at[pl.ds(rs, ch)], rows_b.at[pl.ds(0, ch)])

            def accum(i, _):
                r = i // 8
                col = (i % 8) * 16
                rows_a[r, pl.ds(col, 16)] = (rows_a[r, pl.ds(col, 16)]
                                             + rows_b[r, pl.ds(col, 16)])
                return 0

            lax.fori_loop(0, ch * 8, accum, 0)

        def scalemean(i, _):
            r = i // 8
            col = (i % 8) * 16
            rows_a[r, pl.ds(col, 16)] = rows_a[r, pl.ds(col, 16)] * 0.25
            return 0

        lax.fori_loop(0, ch * 8, scalemean, 0)
        pltpu.sync_copy(rows_a.at[pl.ds(0, ch)], out.at[pl.ds(rs, ch)])

    r0 = c * N_NODES + s * ROWS_PER_TILE
    for q in range(ROWS_PER_TILE // B):
        mean_chunk(r0 + q * B, B)
    mean_chunk(r0 + (ROWS_PER_TILE // B) * B, ROWS_PER_TILE % B)

    @pl.when(s < 2)
    def _mean_rem():
        mean_chunk(c * N_NODES + REM_BASE + s * REM, REM)


_mesh = plsc.VectorSubcoreMesh(core_axis_name="c", subcore_axis_name="s")

_f32 = jnp.float32
_i32 = jnp.int32

_kernel_call = functools.partial(
    pl.kernel,
    mesh=_mesh,
    out_type=[jax.ShapeDtypeStruct((NC * N_NODES, DH), _f32)] * 4,
    scratch_types=[
        pltpu.VMEM((G, B), _i32),
        pltpu.VMEM((G, B), _i32),
        pltpu.VMEM((G, B), _f32),
        pltpu.VMEM((B, DH), _f32),
        pltpu.VMEM((B, DH), _f32),
        pltpu.VMEM_SHARED((ACC_ROWS, DH), _f32),
        pltpu.SemaphoreType.DMA,
        pltpu.SemaphoreType.DMA,
        pltpu.SemaphoreType.DMA,
        pltpu.SemaphoreType.DMA,
    ],
)(_sc_kernel)


def kernel(edge_index, edge_values, user_emb, item_emb):
    ego = jnp.concatenate([user_emb, item_emb], axis=0)
    x0 = ego.reshape(N_NODES, NC, DH).transpose(1, 0, 2).reshape(NC * N_NODES, DH)

    pad = E_PAD - N_EDGES
    src = jnp.pad(edge_index[0], (0, pad))
    dst = jnp.pad(edge_index[1], (0, pad), constant_values=N_NODES)
    val = jnp.pad(edge_values, (0, pad))

    src2 = src.reshape(E_PAD // B, B)
    dst2 = dst.reshape(E_PAD // B, B)
    val2 = val.reshape(E_PAD // B, B)

    mean_cs, _e1, _e2, _e3 = _kernel_call(x0, src2, dst2, val2)
    mean = mean_cs.reshape(NC, N_NODES, DH).transpose(1, 0, 2).reshape(N_NODES, D)
    return (mean[:N_USERS], mean[N_USERS:])
